# baseline (speedup 1.0000x reference)
import functools

import jax
import jax.numpy as jnp
from jax import lax
from jax.experimental import pallas as pl
from jax.experimental.pallas import tpu as pltpu
from jax.experimental.pallas import tpu_sc as plsc

B = 4096
MAXN = 64
EMBED = 256
HIDDEN = 512

NEG_INF = float("-inf")

_NC = 2
_NS = 16
_NW = _NC * _NS
_PIECES = 16
_BP = B // _PIECES
_NTOT = _BP + _BP + _BP * MAXN
_PER_W = _NTOT // _NW
_CHUNK = 88
_NCHUNK = _PER_W // _CHUNK
assert _PER_W % _CHUNK == 0 and _PER_W % 8 == 0 and _CHUNK % 8 == 0


def _sc_gather(table, idx_all):
    mesh = plsc.VectorSubcoreMesh(core_axis_name="c", subcore_axis_name="s")

    @functools.partial(
        pl.kernel,
        mesh=mesh,
        out_type=jax.ShapeDtypeStruct((_NTOT, EMBED), table.dtype),
        scratch_types=[
            pltpu.VMEM((_PER_W,), jnp.int32),
            pltpu.VMEM((_CHUNK, EMBED), table.dtype),
            pltpu.VMEM((_CHUNK, EMBED), table.dtype),
            pltpu.SemaphoreType.DMA,
            pltpu.SemaphoreType.DMA,
        ],
    )
    def gather_kernel(table_hbm, idx_hbm, out_hbm, idx_v, rows0, rows1,
                      sem0, sem1):
        wid = lax.axis_index("s") * _NC + lax.axis_index("c")
        base = wid * _PER_W
        pltpu.sync_copy(idx_hbm.at[pl.ds(base, _PER_W)], idx_v)

        def start(t, rows, sem):
            pltpu.async_copy(
                table_hbm.at[idx_v.at[pl.ds(t * _CHUNK, _CHUNK)]], rows, sem)

        def wait(rows, sem):
            pltpu.make_async_copy(
                table_hbm.at[pl.ds(0, _CHUNK)], rows, sem).wait()

        def write(t, rows):
            pltpu.sync_copy(rows, out_hbm.at[pl.ds(base + t * _CHUNK, _CHUNK)])

        start(0, rows0, sem0)

        @pl.loop(0, _NCHUNK - 2, step=2)
        def _(t):
            start(t + 1, rows1, sem1)
            wait(rows0, sem0)
            write(t, rows0)
            start(t + 2, rows0, sem0)
            wait(rows1, sem1)
            write(t + 1, rows1)

        start(_NCHUNK - 1, rows1, sem1)
        wait(rows0, sem0)
        write(_NCHUNK - 2, rows0)
        wait(rows1, sem1)
        write(_NCHUNK - 1, rows1)

    return gather_kernel(table, idx_all)


_R = 128


def _tc_body(cur_ref, tgt_ref, nbr_ref, mask_ref,
             w1c_ref, w1n_ref, b1_ref, w2_ref, b2_ref,
             w3c_ref, w3t_ref, w3n_ref, b3_ref, w4_ref, b4_ref,
             out_ref):
    f32 = jnp.float32
    bf16 = jnp.bfloat16

    mask2 = mask_ref[...]
    mask3 = mask2[:, :, None]
    nbr = nbr_ref[...]
    nbr_b = nbr.astype(bf16)

    msum = jnp.sum(nbr.reshape(_R, MAXN, EMBED) * mask3, axis=1)
    cnt = jnp.maximum(jnp.sum(mask2, axis=1, keepdims=True), 1.0)
    agg = (msum / cnt).astype(bf16)

    cur = cur_ref[...].astype(bf16)
    tgt = tgt_ref[...].astype(bf16)

    h = jnp.dot(cur, w1c_ref[...], preferred_element_type=f32)
    h += jnp.dot(agg, w1n_ref[...], preferred_element_type=f32)
    h = jnp.maximum(h + b1_ref[...], 0.0)
    ctx = jnp.dot(h.astype(bf16), w2_ref[...], preferred_element_type=f32)
    ctx = (ctx + b2_ref[...]).astype(bf16)

    a = jnp.dot(ctx, w3c_ref[...], preferred_element_type=f32)
    a += jnp.dot(tgt, w3t_ref[...], preferred_element_type=f32)
    a = (a + b3_ref[...]).astype(bf16)

    n3 = jnp.dot(nbr_b, w3n_ref[...], preferred_element_type=f32).astype(bf16)
    h2 = jnp.maximum(n3.reshape(_R, MAXN, HIDDEN) + a[:, None, :], 0.0)
    h2 = h2.reshape(_R * MAXN, HIDDEN)
    s = jnp.dot(h2, w4_ref[...], preferred_element_type=f32)
    s2 = s.reshape(_R, MAXN) + b4_ref[...]

    out_ref[...] = jnp.where(mask2 > 0.0, s2, NEG_INF)


def _tc_score(gathered, mask2, w1c, w1n, b1, w2, b2, w3c, w3t, w3n, b3, w4, b4):
    grid = (_BP // _R,)
    nbr_rows = _R * MAXN
    return pl.pallas_call(
        _tc_body,
        grid=grid,
        in_specs=[
            pl.BlockSpec((_R, EMBED),
                         lambda i: (_BP * MAXN // _R + i, 0)),
            pl.BlockSpec((_R, EMBED),
                         lambda i: (_BP * MAXN // _R + _BP // _R + i, 0)),
            pl.BlockSpec((nbr_rows, EMBED), lambda i: (i, 0)),
            pl.BlockSpec((_R, MAXN), lambda i: (i, 0)),
            pl.BlockSpec((EMBED, HIDDEN), lambda i: (0, 0)),
            pl.BlockSpec((EMBED, HIDDEN), lambda i: (0, 0)),
            pl.BlockSpec((1, HIDDEN), lambda i: (0, 0)),
            pl.BlockSpec((HIDDEN, EMBED), lambda i: (0, 0)),
            pl.BlockSpec((1, EMBED), lambda i: (0, 0)),
            pl.BlockSpec((EMBED, HIDDEN), lambda i: (0, 0)),
            pl.BlockSpec((EMBED, HIDDEN), lambda i: (0, 0)),
            pl.BlockSpec((EMBED, HIDDEN), lambda i: (0, 0)),
            pl.BlockSpec((1, HIDDEN), lambda i: (0, 0)),
            pl.BlockSpec((HIDDEN, 1), lambda i: (0, 0)),
            pl.BlockSpec((1, 1), lambda i: (0, 0)),
        ],
        out_specs=pl.BlockSpec((_R, MAXN), lambda i: (i, 0)),
        out_shape=jax.ShapeDtypeStruct((_BP, MAXN), jnp.float32),
    )(gathered, gathered, gathered, mask2,
      w1c, w1n, b1, w2, b2, w3c, w3t, w3n, b3, w4, b4)


def kernel(current_idx, target_idx, neighbor_indices, neighbor_mask,
           table, W1, b1, W2, b2, W3, b3, W4, b4):
    bf16 = jnp.bfloat16

    w1t = W1.T.astype(bf16)
    w1c, w1n = w1t[:EMBED], w1t[EMBED:]
    w3tf = W3.T.astype(bf16)
    w3c = w3tf[:EMBED]
    w3t = w3tf[EMBED:2 * EMBED]
    w3n = w3tf[2 * EMBED:]
    w2 = W2.T.astype(bf16)
    w4 = W4.T.astype(bf16)
    b1r, b2r = b1.reshape(1, HIDDEN), b2.reshape(1, EMBED)
    b3r, b4r = b3.reshape(1, HIDDEN), b4.reshape(1, 1)

    maskf = neighbor_mask.astype(jnp.float32)

    pieces = []
    for p in range(_PIECES):
        sl = slice(p * _BP, (p + 1) * _BP)
        idx_p = jnp.concatenate(
            [neighbor_indices[sl].reshape(-1),
             current_idx[sl], target_idx[sl]], axis=0)
        gathered = _sc_gather(table, idx_p)
        pieces.append(_tc_score(
            gathered, maskf[sl],
            w1c, w1n, b1r, w2, b2r, w3c, w3t, w3n, b3r, w4, b4r))
    return jnp.concatenate(pieces, axis=0)


# speedup vs baseline: 1.1096x; 1.1096x over previous
import functools

import jax
import jax.numpy as jnp
from jax import lax
from jax.experimental import pallas as pl
from jax.experimental.pallas import tpu as pltpu
from jax.experimental.pallas import tpu_sc as plsc

B = 4096
MAXN = 64
EMBED = 256
HIDDEN = 512

NEG_INF = float("-inf")

_NC = 2
_NS = 16
_NW = _NC * _NS
_PIECES = 8
_BP = B // _PIECES
_NTOT = _BP + _BP + _BP * MAXN
_PER_W = _NTOT // _NW
_CHUNK = 88
_NCHUNK = _PER_W // _CHUNK
assert _PER_W % _CHUNK == 0 and _PER_W % 8 == 0 and _CHUNK % 8 == 0


def _sc_gather(table, idx_all):
    mesh = plsc.VectorSubcoreMesh(core_axis_name="c", subcore_axis_name="s")

    @functools.partial(
        pl.kernel,
        mesh=mesh,
        out_type=jax.ShapeDtypeStruct((_NTOT, EMBED), table.dtype),
        scratch_types=[
            pltpu.VMEM((_PER_W,), jnp.int32),
            pltpu.VMEM((_CHUNK, EMBED), table.dtype),
            pltpu.VMEM((_CHUNK, EMBED), table.dtype),
            pltpu.SemaphoreType.DMA,
            pltpu.SemaphoreType.DMA,
        ],
    )
    def gather_kernel(table_hbm, idx_hbm, out_hbm, idx_v, rows0, rows1,
                      sem0, sem1):
        wid = lax.axis_index("s") * _NC + lax.axis_index("c")
        base = wid * _PER_W
        pltpu.sync_copy(idx_hbm.at[pl.ds(base, _PER_W)], idx_v)

        def start(t, rows, sem):
            pltpu.async_copy(
                table_hbm.at[idx_v.at[pl.ds(t * _CHUNK, _CHUNK)]], rows, sem)

        def wait(rows, sem):
            pltpu.make_async_copy(
                table_hbm.at[pl.ds(0, _CHUNK)], rows, sem).wait()

        def write(t, rows):
            pltpu.sync_copy(rows, out_hbm.at[pl.ds(base + t * _CHUNK, _CHUNK)])

        start(0, rows0, sem0)

        @pl.loop(0, _NCHUNK - 2, step=2)
        def _(t):
            start(t + 1, rows1, sem1)
            wait(rows0, sem0)
            write(t, rows0)
            start(t + 2, rows0, sem0)
            wait(rows1, sem1)
            write(t + 1, rows1)

        start(_NCHUNK - 1, rows1, sem1)
        wait(rows0, sem0)
        write(_NCHUNK - 2, rows0)
        wait(rows1, sem1)
        write(_NCHUNK - 1, rows1)

    return gather_kernel(table, idx_all)


_R = 128


def _tc_body(cur_ref, tgt_ref, nbr_ref, mask_ref,
             w1c_ref, w1n_ref, b1_ref, w2_ref, b2_ref,
             w3c_ref, w3t_ref, w3n_ref, b3_ref, w4_ref, b4_ref,
             out_ref):
    f32 = jnp.float32
    bf16 = jnp.bfloat16

    mask2 = mask_ref[...]
    mask3 = mask2[:, :, None]
    nbr = nbr_ref[...]
    nbr_b = nbr.astype(bf16)

    msum = jnp.sum(nbr.reshape(_R, MAXN, EMBED) * mask3, axis=1)
    cnt = jnp.maximum(jnp.sum(mask2, axis=1, keepdims=True), 1.0)
    agg = (msum / cnt).astype(bf16)

    cur = cur_ref[...].astype(bf16)
    tgt = tgt_ref[...].astype(bf16)

    h = jnp.dot(cur, w1c_ref[...], preferred_element_type=f32)
    h += jnp.dot(agg, w1n_ref[...], preferred_element_type=f32)
    h = jnp.maximum(h + b1_ref[...], 0.0)
    ctx = jnp.dot(h.astype(bf16), w2_ref[...], preferred_element_type=f32)
    ctx = (ctx + b2_ref[...]).astype(bf16)

    a = jnp.dot(ctx, w3c_ref[...], preferred_element_type=f32)
    a += jnp.dot(tgt, w3t_ref[...], preferred_element_type=f32)
    a = (a + b3_ref[...]).astype(bf16)

    n3 = jnp.dot(nbr_b, w3n_ref[...], preferred_element_type=f32).astype(bf16)
    h2 = jnp.maximum(n3.reshape(_R, MAXN, HIDDEN) + a[:, None, :], 0.0)
    h2 = h2.reshape(_R * MAXN, HIDDEN)
    s = jnp.dot(h2, w4_ref[...], preferred_element_type=f32)
    s2 = s.reshape(_R, MAXN) + b4_ref[...]

    out_ref[...] = jnp.where(mask2 > 0.0, s2, NEG_INF)


def _tc_score(gathered, mask2, w1c, w1n, b1, w2, b2, w3c, w3t, w3n, b3, w4, b4):
    grid = (_BP // _R,)
    nbr_rows = _R * MAXN
    return pl.pallas_call(
        _tc_body,
        grid=grid,
        in_specs=[
            pl.BlockSpec((_R, EMBED),
                         lambda i: (_BP * MAXN // _R + i, 0)),
            pl.BlockSpec((_R, EMBED),
                         lambda i: (_BP * MAXN // _R + _BP // _R + i, 0)),
            pl.BlockSpec((nbr_rows, EMBED), lambda i: (i, 0)),
            pl.BlockSpec((_R, MAXN), lambda i: (i, 0)),
            pl.BlockSpec((EMBED, HIDDEN), lambda i: (0, 0)),
            pl.BlockSpec((EMBED, HIDDEN), lambda i: (0, 0)),
            pl.BlockSpec((1, HIDDEN), lambda i: (0, 0)),
            pl.BlockSpec((HIDDEN, EMBED), lambda i: (0, 0)),
            pl.BlockSpec((1, EMBED), lambda i: (0, 0)),
            pl.BlockSpec((EMBED, HIDDEN), lambda i: (0, 0)),
            pl.BlockSpec((EMBED, HIDDEN), lambda i: (0, 0)),
            pl.BlockSpec((EMBED, HIDDEN), lambda i: (0, 0)),
            pl.BlockSpec((1, HIDDEN), lambda i: (0, 0)),
            pl.BlockSpec((HIDDEN, 1), lambda i: (0, 0)),
            pl.BlockSpec((1, 1), lambda i: (0, 0)),
        ],
        out_specs=pl.BlockSpec((_R, MAXN), lambda i: (i, 0)),
        out_shape=jax.ShapeDtypeStruct((_BP, MAXN), jnp.float32),
    )(gathered, gathered, gathered, mask2,
      w1c, w1n, b1, w2, b2, w3c, w3t, w3n, b3, w4, b4)


def kernel(current_idx, target_idx, neighbor_indices, neighbor_mask,
           table, W1, b1, W2, b2, W3, b3, W4, b4):
    bf16 = jnp.bfloat16

    w1t = W1.T.astype(bf16)
    w1c, w1n = w1t[:EMBED], w1t[EMBED:]
    w3tf = W3.T.astype(bf16)
    w3c = w3tf[:EMBED]
    w3t = w3tf[EMBED:2 * EMBED]
    w3n = w3tf[2 * EMBED:]
    w2 = W2.T.astype(bf16)
    w4 = W4.T.astype(bf16)
    b1r, b2r = b1.reshape(1, HIDDEN), b2.reshape(1, EMBED)
    b3r, b4r = b3.reshape(1, HIDDEN), b4.reshape(1, 1)

    maskf = neighbor_mask.astype(jnp.float32)

    pieces = []
    for p in range(_PIECES):
        sl = slice(p * _BP, (p + 1) * _BP)
        idx_p = jnp.concatenate(
            [neighbor_indices[sl].reshape(-1),
             current_idx[sl], target_idx[sl]], axis=0)
        gathered = _sc_gather(table, idx_p)
        pieces.append(_tc_score(
            gathered, maskf[sl],
            w1c, w1n, b1r, w2, b2r, w3c, w3t, w3n, b3r, w4, b4r))
    return jnp.concatenate(pieces, axis=0)
